# Initial kernel scaffold; baseline (speedup 1.0000x reference)
import jax
import jax.numpy as jnp
from jax.experimental import pallas as pl


def kernel(x, tok_table, pos_table, W, b):
    raise NotImplementedError("write your pallas kernel here")


# speedup vs baseline: 5.6677x; 5.6677x over previous
import functools

import jax
import jax.numpy as jnp
from jax import lax
from jax.experimental import pallas as pl
from jax.experimental.pallas import tpu as pltpu

VOCAB = 65
T = 8
BBLK = 1024


def _body(x_ref, tok_ref, pos_ref, w_ref, b_ref, o_ref):
    L = jnp.dot(tok_ref[...], w_ref[...], preferred_element_type=jnp.float32)
    L = L + b_ref[...]
    P = jnp.dot(pos_ref[...], w_ref[...], preferred_element_type=jnp.float32)

    xb = x_ref[...].astype(jnp.int32)
    iota = lax.broadcasted_iota(jnp.int32, (BBLK, T, VOCAB), 2)
    oh = (xb[:, :, None] == iota).astype(jnp.float32)
    out = jnp.dot(
        oh.reshape(BBLK * T, VOCAB), L, preferred_element_type=jnp.float32
    ).reshape(BBLK, T, VOCAB)
    o_ref[...] = out + P[None, :, :]


@jax.jit
def kernel(x, tok_table, pos_table, W, b):
    B, t = x.shape
    grid = (B // BBLK,)
    out = pl.pallas_call(
        _body,
        grid=grid,
        in_specs=[
            pl.BlockSpec((BBLK, t), lambda i: (i, 0)),
            pl.BlockSpec((VOCAB, tok_table.shape[1]), lambda i: (0, 0)),
            pl.BlockSpec((t, pos_table.shape[1]), lambda i: (0, 0)),
            pl.BlockSpec(W.shape, lambda i: (0, 0)),
            pl.BlockSpec((1, VOCAB), lambda i: (0, 0)),
        ],
        out_specs=pl.BlockSpec((BBLK, t, VOCAB), lambda i: (i, 0, 0)),
        out_shape=jax.ShapeDtypeStruct((B, t, VOCAB), jnp.float32),
    )(x, tok_table, pos_table, W, b.reshape(1, VOCAB))
    return out
